# baseline (speedup 1.0000x reference)
import functools

import jax
import jax.numpy as jnp
from jax import lax
from jax.experimental import pallas as pl
from jax.experimental.pallas import tpu as pltpu
from jax.experimental.pallas import tpu_sc as plsc

E = 320000
DIM = 128
V0, V1, V2 = 5, 6, 2
NT = V0 * V1 * V2
NC, NS, L = 2, 16, 16
NW = NC * NS

S = 197120
PER_W = S // NW
CH = 80
NCHUNK = PER_W // CH
NPAIR = (NCHUNK - 1) // 2

TPAD = 64
B = 1024
NB = (E - S) // B


def _sc_part(message, a0, a1, a2, emb0, emb1, emb2):
    mesh = plsc.VectorSubcoreMesh(core_axis_name="c", subcore_axis_name="s")

    @functools.partial(
        pl.kernel,
        out_type=jax.ShapeDtypeStruct((S, DIM), jnp.float32),
        mesh=mesh,
        scratch_types=[
            pltpu.VMEM((NT, DIM), jnp.float32),
            pltpu.VMEM((V0, DIM), jnp.float32),
            pltpu.VMEM((V1, DIM), jnp.float32),
            pltpu.VMEM((V2, DIM), jnp.float32),
            pltpu.VMEM((CH, DIM), jnp.float32),
            pltpu.VMEM((CH, DIM), jnp.float32),
            pltpu.VMEM((CH, DIM), jnp.float32),
            pltpu.VMEM((CH, DIM), jnp.float32),
            pltpu.VMEM((CH,), jnp.int32),
            pltpu.VMEM((CH,), jnp.int32),
            pltpu.VMEM((CH,), jnp.int32),
            pltpu.VMEM((CH,), jnp.int32),
            pltpu.VMEM((CH,), jnp.int32),
            pltpu.VMEM((CH,), jnp.int32),
            pltpu.SemaphoreType.DMA,
            pltpu.SemaphoreType.DMA,
            pltpu.SemaphoreType.DMA,
            pltpu.SemaphoreType.DMA,
        ],
    )
    def k(msg_hbm, a0_hbm, a1_hbm, a2_hbm, e0_hbm, e1_hbm, e2_hbm, out_hbm,
          tbl, e0v, e1v, e2v, mbuf0, mbuf1, obuf0, obuf1,
          i00, i10, i20, i01, i11, i21,
          insem0, insem1, outsem0, outsem1):
        mbufs = (mbuf0, mbuf1)
        obufs = (obuf0, obuf1)
        ibufs = ((i00, i10, i20), (i01, i11, i21))
        insems = (insem0, insem1)
        outsems = (outsem0, outsem1)

        wid = lax.axis_index("s") * NC + lax.axis_index("c")
        base = wid * PER_W

        pltpu.sync_copy(e0_hbm, e0v)
        pltpu.sync_copy(e1_hbm, e1v)
        pltpu.sync_copy(e2_hbm, e2v)

        def build(cc, carry):
            k0 = cc % V0
            k1 = (cc // V0) % V1
            k2 = cc // (V0 * V1)
            for j in range(DIM // L):
                sl = pl.ds(j * L, L)
                tbl[cc, sl] = e0v[k0, sl] + e1v[k1, sl] + e2v[k2, sl]
            return carry

        lax.fori_loop(0, NT, build, 0)

        def start_in(t, b):
            rows = pl.ds(base + t * CH, CH)
            pltpu.async_copy(msg_hbm.at[rows], mbufs[b], insems[b])
            pltpu.async_copy(a0_hbm.at[rows], ibufs[b][0], insems[b])
            pltpu.async_copy(a1_hbm.at[rows], ibufs[b][1], insems[b])
            pltpu.async_copy(a2_hbm.at[rows], ibufs[b][2], insems[b])

        def wait_in(t, b):
            rows = pl.ds(base + t * CH, CH)
            pltpu.make_async_copy(msg_hbm.at[rows], mbufs[b], insems[b]).wait()
            pltpu.make_async_copy(a0_hbm.at[rows], ibufs[b][0], insems[b]).wait()
            pltpu.make_async_copy(a1_hbm.at[rows], ibufs[b][1], insems[b]).wait()
            pltpu.make_async_copy(a2_hbm.at[rows], ibufs[b][2], insems[b]).wait()

        def start_out(t, b):
            rows = pl.ds(base + t * CH, CH)
            pltpu.async_copy(obufs[b], out_hbm.at[rows], outsems[b])

        def wait_out(t, b):
            rows = pl.ds(base + t * CH, CH)
            pltpu.make_async_copy(obufs[b], out_hbm.at[rows], outsems[b]).wait()

        def compute(b):
            mb, ob = mbufs[b], obufs[b]
            i0b, i1b, i2b = ibufs[b]

            def grp(g, c2):
                sl16 = pl.ds(g * L, L)
                civ = i0b[sl16] + V0 * i1b[sl16] + (V0 * V1) * i2b[sl16]
                for lane in range(L):
                    c = civ[lane]
                    e = g * L + lane
                    ms = [mb[e, pl.ds(j * L, L)] for j in range(DIM // L)]
                    ts = [tbl[c, pl.ds(j * L, L)] for j in range(DIM // L)]
                    for j in range(DIM // L):
                        ob[e, pl.ds(j * L, L)] = jnp.maximum(ms[j] + ts[j], 0.0)
                return c2

            lax.fori_loop(0, CH // L, grp, 0)

        start_in(0, 0)
        start_in(1, 1)

        def pair_body(p, carry):
            t0 = 2 * p
            wait_in(t0, 0)

            @pl.when(p >= 1)
            def _():
                wait_out(t0 - 2, 0)

            compute(0)
            start_out(t0, 0)
            start_in(t0 + 2, 0)

            wait_in(t0 + 1, 1)

            @pl.when(p >= 1)
            def _():
                wait_out(t0 - 1, 1)

            compute(1)
            start_out(t0 + 1, 1)

            @pl.when(p <= NPAIR - 2)
            def _():
                start_in(t0 + 3, 1)

            return carry

        lax.fori_loop(0, NPAIR, pair_body, 0)

        t_last = NCHUNK - 1
        wait_in(t_last, 0)
        wait_out(t_last - 2, 0)
        compute(0)
        start_out(t_last, 0)
        wait_out(t_last - 1, 1)
        wait_out(t_last, 0)

    return k(message, a0, a1, a2, emb0, emb1, emb2)


def _tc_body(a0_ref, a1_ref, a2_ref, e0_ref, e1_ref, e2_ref, msg_ref, out_ref):
    c = (V1 * V2) * a0_ref[0, 0, :] + V2 * a1_ref[0, 0, :] + a2_ref[0, 0, :]
    onehot = (c[:, None] == lax.broadcasted_iota(jnp.int32, (1, TPAD), 1)
              ).astype(jnp.float32)
    e0 = e0_ref[...]
    e1 = e1_ref[...]
    e2 = e2_ref[...]
    t = (e0[:, None, None, :] + e1[None, :, None, :]
         + e2[None, None, :, :]).reshape(NT, DIM)
    t64 = jnp.concatenate([t, jnp.zeros((TPAD - NT, DIM), jnp.float32)], axis=0)
    bond = jnp.dot(onehot, t64, preferred_element_type=jnp.float32)
    out_ref[...] = jnp.maximum(msg_ref[...] + bond, 0.0)


def _tc_part(message, a0, a1, a2, emb0, emb1, emb2):
    a0r = a0.reshape(NB, 1, B)
    a1r = a1.reshape(NB, 1, B)
    a2r = a2.reshape(NB, 1, B)
    full = lambda shape: pl.BlockSpec(shape, lambda i: (0,) * len(shape))
    return pl.pallas_call(
        _tc_body,
        grid=(NB,),
        in_specs=[
            pl.BlockSpec((1, 1, B), lambda i: (i, 0, 0)),
            pl.BlockSpec((1, 1, B), lambda i: (i, 0, 0)),
            pl.BlockSpec((1, 1, B), lambda i: (i, 0, 0)),
            full((V0, DIM)),
            full((V1, DIM)),
            full((V2, DIM)),
            pl.BlockSpec((B, DIM), lambda i: (i, 0)),
        ],
        out_specs=pl.BlockSpec((B, DIM), lambda i: (i, 0)),
        out_shape=jax.ShapeDtypeStruct((E - S, DIM), jnp.float32),
    )(a0r, a1r, a2r, emb0, emb1, emb2, message)


def kernel(message, attrs, emb0, emb1, emb2):
    a = attrs.astype(jnp.int32)
    a0, a1, a2 = a[:, 0], a[:, 1], a[:, 2]
    sc_out = _sc_part(message[:S], a0[:S], a1[:S], a2[:S], emb0, emb1, emb2)
    tc_out = _tc_part(message[S:], a0[S:], a1[S:], a2[S:], emb0, emb1, emb2)
    return jnp.concatenate([sc_out, tc_out], axis=0)


# speedup vs baseline: 1.1599x; 1.1599x over previous
import functools

import jax
import jax.numpy as jnp
from jax import lax
from jax.experimental import pallas as pl
from jax.experimental.pallas import tpu as pltpu
from jax.experimental.pallas import tpu_sc as plsc

E = 320000
DIM = 128
V0, V1, V2 = 5, 6, 2
NT = V0 * V1 * V2
NC, NS, L = 2, 16, 16
NW = NC * NS

S = 197120
PER_W = S // NW
CH = 80
NCHUNK = PER_W // CH
NPAIR = (NCHUNK - 1) // 2

TPAD = 64
B = 512
NB = (E - S) // B
OFFB = S // B


def _sc_part(message, a0, a1, a2, emb0, emb1, emb2):
    mesh = plsc.VectorSubcoreMesh(core_axis_name="c", subcore_axis_name="s")

    @functools.partial(
        pl.kernel,
        out_type=jax.ShapeDtypeStruct((S, DIM), jnp.float32),
        mesh=mesh,
        scratch_types=[
            pltpu.VMEM((NT, DIM), jnp.float32),
            pltpu.VMEM((V0, DIM), jnp.float32),
            pltpu.VMEM((V1, DIM), jnp.float32),
            pltpu.VMEM((V2, DIM), jnp.float32),
            pltpu.VMEM((CH, DIM), jnp.float32),
            pltpu.VMEM((CH, DIM), jnp.float32),
            pltpu.VMEM((CH, DIM), jnp.float32),
            pltpu.VMEM((CH, DIM), jnp.float32),
            pltpu.VMEM((CH,), jnp.int32),
            pltpu.VMEM((CH,), jnp.int32),
            pltpu.VMEM((CH,), jnp.int32),
            pltpu.VMEM((CH,), jnp.int32),
            pltpu.VMEM((CH,), jnp.int32),
            pltpu.VMEM((CH,), jnp.int32),
            pltpu.SemaphoreType.DMA,
            pltpu.SemaphoreType.DMA,
            pltpu.SemaphoreType.DMA,
            pltpu.SemaphoreType.DMA,
        ],
    )
    def k(msg_hbm, a0_hbm, a1_hbm, a2_hbm, e0_hbm, e1_hbm, e2_hbm, out_hbm,
          tbl, e0v, e1v, e2v, mbuf0, mbuf1, obuf0, obuf1,
          i00, i10, i20, i01, i11, i21,
          insem0, insem1, outsem0, outsem1):
        mbufs = (mbuf0, mbuf1)
        obufs = (obuf0, obuf1)
        ibufs = ((i00, i10, i20), (i01, i11, i21))
        insems = (insem0, insem1)
        outsems = (outsem0, outsem1)

        wid = lax.axis_index("s") * NC + lax.axis_index("c")
        base = wid * PER_W

        pltpu.sync_copy(e0_hbm, e0v)
        pltpu.sync_copy(e1_hbm, e1v)
        pltpu.sync_copy(e2_hbm, e2v)

        def build(cc, carry):
            k0 = cc % V0
            k1 = (cc // V0) % V1
            k2 = cc // (V0 * V1)
            for j in range(DIM // L):
                sl = pl.ds(j * L, L)
                tbl[cc, sl] = e0v[k0, sl] + e1v[k1, sl] + e2v[k2, sl]
            return carry

        lax.fori_loop(0, NT, build, 0)

        def start_in(t, b):
            rows = pl.ds(base + t * CH, CH)
            pltpu.async_copy(msg_hbm.at[rows], mbufs[b], insems[b])
            pltpu.async_copy(a0_hbm.at[rows], ibufs[b][0], insems[b])
            pltpu.async_copy(a1_hbm.at[rows], ibufs[b][1], insems[b])
            pltpu.async_copy(a2_hbm.at[rows], ibufs[b][2], insems[b])

        def wait_in(t, b):
            rows = pl.ds(base + t * CH, CH)
            pltpu.make_async_copy(msg_hbm.at[rows], mbufs[b], insems[b]).wait()
            pltpu.make_async_copy(a0_hbm.at[rows], ibufs[b][0], insems[b]).wait()
            pltpu.make_async_copy(a1_hbm.at[rows], ibufs[b][1], insems[b]).wait()
            pltpu.make_async_copy(a2_hbm.at[rows], ibufs[b][2], insems[b]).wait()

        def start_out(t, b):
            rows = pl.ds(base + t * CH, CH)
            pltpu.async_copy(obufs[b], out_hbm.at[rows], outsems[b])

        def wait_out(t, b):
            rows = pl.ds(base + t * CH, CH)
            pltpu.make_async_copy(obufs[b], out_hbm.at[rows], outsems[b]).wait()

        def compute(b):
            mb, ob = mbufs[b], obufs[b]
            i0b, i1b, i2b = ibufs[b]

            def grp(g, c2):
                sl16 = pl.ds(g * L, L)
                civ = i0b[sl16] + V0 * i1b[sl16] + (V0 * V1) * i2b[sl16]
                for lane in range(L):
                    c = civ[lane]
                    e = g * L + lane
                    ms = [mb[e, pl.ds(j * L, L)] for j in range(DIM // L)]
                    ts = [tbl[c, pl.ds(j * L, L)] for j in range(DIM // L)]
                    for j in range(DIM // L):
                        ob[e, pl.ds(j * L, L)] = jnp.maximum(ms[j] + ts[j], 0.0)
                return c2

            lax.fori_loop(0, CH // L, grp, 0)

        start_in(0, 0)
        start_in(1, 1)

        def pair_body(p, carry):
            t0 = 2 * p
            wait_in(t0, 0)

            @pl.when(p >= 1)
            def _():
                wait_out(t0 - 2, 0)

            compute(0)
            start_out(t0, 0)
            start_in(t0 + 2, 0)

            wait_in(t0 + 1, 1)

            @pl.when(p >= 1)
            def _():
                wait_out(t0 - 1, 1)

            compute(1)
            start_out(t0 + 1, 1)

            @pl.when(p <= NPAIR - 2)
            def _():
                start_in(t0 + 3, 1)

            return carry

        lax.fori_loop(0, NPAIR, pair_body, 0)

        t_last = NCHUNK - 1
        wait_in(t_last, 0)
        wait_out(t_last - 2, 0)
        compute(0)
        start_out(t_last, 0)
        wait_out(t_last - 1, 1)
        wait_out(t_last, 0)

    return k(message, a0, a1, a2, emb0, emb1, emb2)


def _tc_body(a0_ref, a1_ref, a2_ref, e0_ref, e1_ref, e2_ref, msg_ref, out_ref):
    c = (V1 * V2) * a0_ref[0, 0, :] + V2 * a1_ref[0, 0, :] + a2_ref[0, 0, :]
    onehot = (c[:, None] == lax.broadcasted_iota(jnp.int32, (1, TPAD), 1)
              ).astype(jnp.float32)
    e0 = e0_ref[...]
    e1 = e1_ref[...]
    e2 = e2_ref[...]
    t = (e0[:, None, None, :] + e1[None, :, None, :]
         + e2[None, None, :, :]).reshape(NT, DIM)
    t64 = jnp.concatenate([t, jnp.zeros((TPAD - NT, DIM), jnp.float32)], axis=0)
    bond = jnp.dot(onehot, t64, preferred_element_type=jnp.float32)
    out_ref[...] = jnp.maximum(msg_ref[...] + bond, 0.0)


def _tc_part(message, a0, a1, a2, emb0, emb1, emb2):
    a0r = a0.reshape(E // B, 1, B)
    a1r = a1.reshape(E // B, 1, B)
    a2r = a2.reshape(E // B, 1, B)
    full = lambda shape: pl.BlockSpec(shape, lambda i: (0,) * len(shape))
    return pl.pallas_call(
        _tc_body,
        grid=(NB,),
        in_specs=[
            pl.BlockSpec((1, 1, B), lambda i: (OFFB + i, 0, 0)),
            pl.BlockSpec((1, 1, B), lambda i: (OFFB + i, 0, 0)),
            pl.BlockSpec((1, 1, B), lambda i: (OFFB + i, 0, 0)),
            full((V0, DIM)),
            full((V1, DIM)),
            full((V2, DIM)),
            pl.BlockSpec((B, DIM), lambda i: (OFFB + i, 0)),
        ],
        out_specs=pl.BlockSpec((B, DIM), lambda i: (i, 0)),
        out_shape=jax.ShapeDtypeStruct((E - S, DIM), jnp.float32),
    )(a0r, a1r, a2r, emb0, emb1, emb2, message)


def kernel(message, attrs, emb0, emb1, emb2):
    a = attrs.astype(jnp.int32)
    a0, a1, a2 = a[:, 0], a[:, 1], a[:, 2]
    sc_out = _sc_part(message, a0, a1, a2, emb0, emb1, emb2)
    tc_out = _tc_part(message, a0, a1, a2, emb0, emb1, emb2)
    return jnp.concatenate([sc_out, tc_out], axis=0)


# speedup vs baseline: 2.1193x; 1.8271x over previous
import functools

import jax
import jax.numpy as jnp
from jax import lax
from jax.experimental import pallas as pl
from jax.experimental.pallas import tpu as pltpu
from jax.experimental.pallas import tpu_sc as plsc

E = 320000
DIM = 128
V0, V1, V2 = 5, 6, 2
NT = V0 * V1 * V2
NC, NS, L = 2, 16, 16
NW = NC * NS
PER_W = E // NW
CH = 80
NCHUNK = PER_W // CH
NPAIR = (NCHUNK - 1) // 2


def kernel(message, attrs, emb0, emb1, emb2):
    a = attrs.astype(jnp.int32)
    a0, a1, a2 = a[:, 0], a[:, 1], a[:, 2]
    mesh = plsc.VectorSubcoreMesh(core_axis_name="c", subcore_axis_name="s")

    @functools.partial(
        pl.kernel,
        out_type=jax.ShapeDtypeStruct((E, DIM), jnp.float32),
        mesh=mesh,
        scratch_types=[
            pltpu.VMEM((NT, DIM), jnp.float32),
            pltpu.VMEM((V0, DIM), jnp.float32),
            pltpu.VMEM((V1, DIM), jnp.float32),
            pltpu.VMEM((V2, DIM), jnp.float32),
            pltpu.VMEM((CH, DIM), jnp.float32),
            pltpu.VMEM((CH, DIM), jnp.float32),
            pltpu.VMEM((CH, DIM), jnp.float32),
            pltpu.VMEM((CH, DIM), jnp.float32),
            pltpu.VMEM((CH,), jnp.int32),
            pltpu.VMEM((CH,), jnp.int32),
            pltpu.VMEM((CH,), jnp.int32),
            pltpu.VMEM((CH,), jnp.int32),
            pltpu.VMEM((CH,), jnp.int32),
            pltpu.VMEM((CH,), jnp.int32),
            pltpu.SemaphoreType.DMA,
            pltpu.SemaphoreType.DMA,
            pltpu.SemaphoreType.DMA,
            pltpu.SemaphoreType.DMA,
        ],
    )
    def k(msg_hbm, a0_hbm, a1_hbm, a2_hbm, e0_hbm, e1_hbm, e2_hbm, out_hbm,
          tbl, e0v, e1v, e2v, mbuf0, mbuf1, obuf0, obuf1,
          i00, i10, i20, i01, i11, i21,
          insem0, insem1, outsem0, outsem1):
        mbufs = (mbuf0, mbuf1)
        obufs = (obuf0, obuf1)
        ibufs = ((i00, i10, i20), (i01, i11, i21))
        insems = (insem0, insem1)
        outsems = (outsem0, outsem1)

        wid = lax.axis_index("s") * NC + lax.axis_index("c")
        base = wid * PER_W

        pltpu.sync_copy(e0_hbm, e0v)
        pltpu.sync_copy(e1_hbm, e1v)
        pltpu.sync_copy(e2_hbm, e2v)

        def build(cc, carry):
            k0 = cc % V0
            k1 = (cc // V0) % V1
            k2 = cc // (V0 * V1)
            for j in range(DIM // L):
                sl = pl.ds(j * L, L)
                tbl[cc, sl] = e0v[k0, sl] + e1v[k1, sl] + e2v[k2, sl]
            return carry

        lax.fori_loop(0, NT, build, 0)

        def start_in(t, b):
            rows = pl.ds(base + t * CH, CH)
            pltpu.async_copy(msg_hbm.at[rows], mbufs[b], insems[b])
            pltpu.async_copy(a0_hbm.at[rows], ibufs[b][0], insems[b])
            pltpu.async_copy(a1_hbm.at[rows], ibufs[b][1], insems[b])
            pltpu.async_copy(a2_hbm.at[rows], ibufs[b][2], insems[b])

        def wait_in(t, b):
            rows = pl.ds(base + t * CH, CH)
            pltpu.make_async_copy(msg_hbm.at[rows], mbufs[b], insems[b]).wait()
            pltpu.make_async_copy(a0_hbm.at[rows], ibufs[b][0], insems[b]).wait()
            pltpu.make_async_copy(a1_hbm.at[rows], ibufs[b][1], insems[b]).wait()
            pltpu.make_async_copy(a2_hbm.at[rows], ibufs[b][2], insems[b]).wait()

        def start_out(t, b):
            rows = pl.ds(base + t * CH, CH)
            pltpu.async_copy(obufs[b], out_hbm.at[rows], outsems[b])

        def wait_out(t, b):
            rows = pl.ds(base + t * CH, CH)
            pltpu.make_async_copy(obufs[b], out_hbm.at[rows], outsems[b]).wait()

        def compute(b):
            mb, ob = mbufs[b], obufs[b]
            i0b, i1b, i2b = ibufs[b]

            def grp(g, c2):
                sl16 = pl.ds(g * L, L)
                civ = i0b[sl16] + V0 * i1b[sl16] + (V0 * V1) * i2b[sl16]
                for lane in range(L):
                    c = civ[lane]
                    e = g * L + lane
                    ms = [mb[e, pl.ds(j * L, L)] for j in range(DIM // L)]
                    ts = [tbl[c, pl.ds(j * L, L)] for j in range(DIM // L)]
                    for j in range(DIM // L):
                        ob[e, pl.ds(j * L, L)] = jnp.maximum(ms[j] + ts[j], 0.0)
                return c2

            lax.fori_loop(0, CH // L, grp, 0)

        start_in(0, 0)
        start_in(1, 1)

        def pair_body(p, carry):
            t0 = 2 * p
            wait_in(t0, 0)

            @pl.when(p >= 1)
            def _():
                wait_out(t0 - 2, 0)

            compute(0)
            start_out(t0, 0)
            start_in(t0 + 2, 0)
            wait_in(t0 + 1, 1)

            @pl.when(p >= 1)
            def _():
                wait_out(t0 - 1, 1)

            compute(1)
            start_out(t0 + 1, 1)

            @pl.when(p <= NPAIR - 2)
            def _():
                start_in(t0 + 3, 1)

            return carry

        lax.fori_loop(0, NPAIR, pair_body, 0)

        t_last = NCHUNK - 1
        wait_in(t_last, 0)
        wait_out(t_last - 2, 0)
        compute(0)
        start_out(t_last, 0)
        wait_out(t_last - 1, 1)
        wait_out(t_last, 0)

    return k(message, a0, a1, a2, emb0, emb1, emb2)


# speedup vs baseline: 2.4928x; 1.1762x over previous
import functools

import jax
import jax.numpy as jnp
from jax import lax
from jax.experimental import pallas as pl
from jax.experimental.pallas import tpu as pltpu
from jax.experimental.pallas import tpu_sc as plsc

E = 320000
DIM = 128
V0, V1, V2 = 5, 6, 2
NT = V0 * V1 * V2
NC, NS, L = 2, 16, 16
NW = NC * NS
PER_W = E // NW
CH = 80
NCHUNK = PER_W // CH
NBUF = 4
NQUAD = (NCHUNK - 1) // NBUF


def kernel(message, attrs, emb0, emb1, emb2):
    a = attrs.astype(jnp.int32)
    a0, a1, a2 = a[:, 0], a[:, 1], a[:, 2]
    mesh = plsc.VectorSubcoreMesh(core_axis_name="c", subcore_axis_name="s")

    row_bufs = [pltpu.VMEM((CH, DIM), jnp.float32) for _ in range(2 * NBUF)]
    idx_bufs = [pltpu.VMEM((CH,), jnp.int32) for _ in range(3 * NBUF)]
    sems = [pltpu.SemaphoreType.DMA for _ in range(2 * NBUF)]

    @functools.partial(
        pl.kernel,
        out_type=jax.ShapeDtypeStruct((E, DIM), jnp.float32),
        mesh=mesh,
        scratch_types=[
            pltpu.VMEM((NT, DIM), jnp.float32),
            pltpu.VMEM((V0, DIM), jnp.float32),
            pltpu.VMEM((V1, DIM), jnp.float32),
            pltpu.VMEM((V2, DIM), jnp.float32),
        ] + row_bufs + idx_bufs + sems,
    )
    def k(msg_hbm, a0_hbm, a1_hbm, a2_hbm, e0_hbm, e1_hbm, e2_hbm, out_hbm,
          tbl, e0v, e1v, e2v, *rest):
        mbufs = rest[0:NBUF]
        obufs = rest[NBUF:2 * NBUF]
        ibufs = tuple(rest[2 * NBUF + 3 * b: 2 * NBUF + 3 * b + 3]
                      for b in range(NBUF))
        insems = rest[5 * NBUF: 6 * NBUF]
        outsems = rest[6 * NBUF: 7 * NBUF]

        wid = lax.axis_index("s") * NC + lax.axis_index("c")
        base = wid * PER_W

        pltpu.sync_copy(e0_hbm, e0v)
        pltpu.sync_copy(e1_hbm, e1v)
        pltpu.sync_copy(e2_hbm, e2v)

        def build(cc, carry):
            k0 = cc % V0
            k1 = (cc // V0) % V1
            k2 = cc // (V0 * V1)
            for j in range(DIM // L):
                sl = pl.ds(j * L, L)
                tbl[cc, sl] = e0v[k0, sl] + e1v[k1, sl] + e2v[k2, sl]
            return carry

        lax.fori_loop(0, NT, build, 0)

        def start_in(t, b):
            rows = pl.ds(base + t * CH, CH)
            pltpu.async_copy(msg_hbm.at[rows], mbufs[b], insems[b])
            pltpu.async_copy(a0_hbm.at[rows], ibufs[b][0], insems[b])
            pltpu.async_copy(a1_hbm.at[rows], ibufs[b][1], insems[b])
            pltpu.async_copy(a2_hbm.at[rows], ibufs[b][2], insems[b])

        def wait_in(t, b):
            rows = pl.ds(base + t * CH, CH)
            pltpu.make_async_copy(msg_hbm.at[rows], mbufs[b], insems[b]).wait()
            pltpu.make_async_copy(a0_hbm.at[rows], ibufs[b][0], insems[b]).wait()
            pltpu.make_async_copy(a1_hbm.at[rows], ibufs[b][1], insems[b]).wait()
            pltpu.make_async_copy(a2_hbm.at[rows], ibufs[b][2], insems[b]).wait()

        def start_out(t, b):
            rows = pl.ds(base + t * CH, CH)
            pltpu.async_copy(obufs[b], out_hbm.at[rows], outsems[b])

        def wait_out(t, b):
            rows = pl.ds(base + t * CH, CH)
            pltpu.make_async_copy(obufs[b], out_hbm.at[rows], outsems[b]).wait()

        def compute(b):
            mb, ob = mbufs[b], obufs[b]
            i0b, i1b, i2b = ibufs[b]

            def grp(g, c2):
                sl16 = pl.ds(g * L, L)
                civ = i0b[sl16] + V0 * i1b[sl16] + (V0 * V1) * i2b[sl16]
                for lane in range(L):
                    c = civ[lane]
                    e = g * L + lane
                    ms = [mb[e, pl.ds(j * L, L)] for j in range(DIM // L)]
                    ts = [tbl[c, pl.ds(j * L, L)] for j in range(DIM // L)]
                    for j in range(DIM // L):
                        ob[e, pl.ds(j * L, L)] = jnp.maximum(ms[j] + ts[j], 0.0)
                return c2

            lax.fori_loop(0, CH // L, grp, 0)

        for b in range(NBUF):
            start_in(b, b)

        def quad_body(p, carry):
            t0 = NBUF * p
            for b in range(NBUF):
                t = t0 + b
                wait_in(t, b)

                @pl.when(p >= 1)
                def _():
                    wait_out(t - NBUF, b)

                compute(b)
                start_out(t, b)

                @pl.when(t + NBUF <= NCHUNK - 1)
                def _():
                    start_in(t + NBUF, b)

            return carry

        lax.fori_loop(0, NQUAD, quad_body, 0)

        t_last = NCHUNK - 1
        wait_in(t_last, 0)
        wait_out(t_last - NBUF, 0)
        compute(0)
        start_out(t_last, 0)
        for b in range(1, NBUF):
            wait_out(t_last - NBUF + b, b)
        wait_out(t_last, 0)

    return k(message, a0, a1, a2, emb0, emb1, emb2)
